# baseline (speedup 1.0000x reference)
import functools

import jax
import jax.numpy as jnp
from jax.experimental import pallas as pl
from jax.experimental.pallas import tpu as pltpu

_TILE = 128
_NEG = -1e30


def _attn_body(x_ref, wl_ref, wr_ref, att_ref, bias_ref, srcb_ref, refb_ref,
               kb_ref, out_ref, xl_ref, xrt_ref, *, heads, out_ch,
               n_half, apply_elu, nbounds):
    i = pl.program_id(0)
    n_tiles = n_half // _TILE

    @pl.when(i == 0)
    def _project():
        xx = x_ref[...]
        xl_ref[...] = jnp.dot(xx, wl_ref[...],
                              preferred_element_type=jnp.float32)
        xrt_ref[...] = jax.lax.dot_general(
            wr_ref[...], xx, (((0,), (1,)), ((), ())),
            preferred_element_type=jnp.float32)

    @pl.when(i > 0)
    def _attend():
        idx = i - 1
        dirn = idx // n_tiles
        j = idx % n_tiles
        dst_base = jnp.where(dirn == 0, n_half + j * _TILE, j * _TILE)
        src_off = jnp.where(dirn == 0, 0, n_half)

        t_row = jax.lax.broadcasted_iota(jnp.int32, (1, _TILE), 1) + j * _TILE
        cnt_t = jnp.zeros((1, _TILE), jnp.int32)
        for b in range(nbounds):
            dstb = jnp.where(dirn == 0, refb_ref[b], srcb_ref[b])
            cnt_t = cnt_t + jnp.where(dstb <= t_row, 1, 0)
        bt_row = cnt_t - 1
        valid_t = (bt_row >= 0) & (bt_row < nbounds - 1)

        k0 = kb_ref[dirn, j, 0]
        k1 = kb_ref[dirn, j, 1]

        def step(k, carry):
            s_base = k * _TILE
            a_full = xl_ref[pl.ds(src_off + s_base, _TILE), :]
            s_col = (jax.lax.broadcasted_iota(jnp.int32, (_TILE, 1), 0)
                     + s_base)
            cnt_s = jnp.zeros((_TILE, 1), jnp.int32)
            for b in range(nbounds):
                srcb = jnp.where(dirn == 0, srcb_ref[b], refb_ref[b])
                cnt_s = cnt_s + jnp.where(srcb <= s_col, 1, 0)
            bs_col = cnt_s - 1
            valid_s = (bs_col >= 0) & (bs_col < nbounds - 1)
            mask = (bs_col == bt_row) & valid_s & valid_t
            maskf = mask.astype(jnp.float32)

            new_carry = []
            for h in range(heads):
                m, d, acc = carry[h]
                a = a_full[:, h * out_ch:(h + 1) * out_ch]
                bT = xrt_ref[h * out_ch:(h + 1) * out_ch,
                             pl.ds(dst_base, _TILE)]
                attc = att_ref[h * out_ch:(h + 1) * out_ch,
                               :].reshape(1, out_ch, 1)
                parts = []
                for sc in range(0, _TILE, 32):
                    a3 = a[sc:sc + 32].reshape(32, out_ch, 1)
                    e = a3 + bT[None, :, :]
                    e = jnp.maximum(e, 0.2 * e) * attc
                    parts.append(jnp.sum(e, axis=1))
                logit = jnp.concatenate(parts, axis=0)
                lm = jnp.where(mask, logit, _NEG)
                tmax = jnp.max(lm, axis=0, keepdims=True)
                m_new = jnp.maximum(m, tmax)
                corr = jnp.exp(m - m_new)
                p = jnp.exp(lm - m_new) * maskf
                d_new = d * corr + jnp.sum(p, axis=0, keepdims=True)
                pv = jax.lax.dot_general(
                    p, a, (((0,), (0,)), ((), ())),
                    preferred_element_type=jnp.float32)
                acc_new = acc * corr.reshape(_TILE, 1) + pv
                new_carry.append((m_new, d_new, acc_new))
            return tuple(new_carry)

        init = tuple((jnp.full((1, _TILE), _NEG, jnp.float32),
                      jnp.zeros((1, _TILE), jnp.float32),
                      jnp.zeros((_TILE, out_ch), jnp.float32))
                     for _ in range(heads))
        carry = jax.lax.fori_loop(k0, k1 + 1, step, init)

        outs = []
        for h in range(heads):
            m, d, acc = carry[h]
            den = jnp.maximum(d.reshape(_TILE, 1), 1e-16)
            outs.append(acc / den)
        val = jnp.concatenate(outs, axis=1) + bias_ref[...]
        if apply_elu:
            val = jnp.where(val > 0, val, jnp.exp(jnp.minimum(val, 0.0)) - 1.0)
        out_ref[pl.ds(dst_base, _TILE), :] = val


def _layer(x, wl, wr, att, bias, srcb, refb, kb, heads, apply_elu):
    n2, _ = x.shape
    hc = wl.shape[1]
    out_ch = hc // heads
    n_half = n2 // 2
    n_tiles = n_half // _TILE
    grid = (1 + 2 * n_tiles,)
    body = functools.partial(
        _attn_body, heads=heads, out_ch=out_ch, n_half=n_half,
        apply_elu=apply_elu, nbounds=srcb.shape[0])
    vmem = pl.BlockSpec(memory_space=pltpu.VMEM)
    smem = pl.BlockSpec(memory_space=pltpu.SMEM)
    return pl.pallas_call(
        body,
        grid=grid,
        in_specs=[vmem, vmem, vmem, vmem, vmem, smem, smem, smem],
        out_specs=vmem,
        out_shape=jax.ShapeDtypeStruct((n2, hc), jnp.float32),
        scratch_shapes=[pltpu.VMEM((n2, hc), jnp.float32),
                        pltpu.VMEM((hc, n2), jnp.float32)],
    )(x, wl, wr, att.reshape(hc, 1), bias.reshape(1, hc), srcb, refb, kb)


def _tile_bounds(srcb, refb, n_dst, n_src):
    nb = srcb.shape[0] - 1

    def one(dstb, sb):
        idxn = jnp.arange(n_dst, dtype=jnp.int32)
        bt = jnp.searchsorted(dstb, idxn, side='right').astype(jnp.int32) - 1
        valid = (bt >= 0) & (bt < nb)
        btt = bt.reshape(-1, _TILE)
        vt = valid.reshape(-1, _TILE)
        blo = jnp.min(jnp.where(vt, btt, nb), axis=1)
        bhi = jnp.max(jnp.where(vt, btt, -1), axis=1)
        any_valid = jnp.any(vt, axis=1)
        s_lo = sb[jnp.clip(blo, 0, nb)]
        s_hi = sb[jnp.clip(bhi + 1, 0, nb)]
        k0 = s_lo // _TILE
        k1 = (s_hi + _TILE - 1) // _TILE - 1
        empty = (~any_valid) | (s_hi <= s_lo)
        k0 = jnp.where(empty, 1, k0)
        k1 = jnp.where(empty, 0, k1)
        return jnp.stack([k0, k1], axis=1)

    return jnp.stack([one(refb, srcb), one(srcb, refb)], 0).astype(jnp.int32)


def kernel(x_s, x_t, src_batch, ref_batch, W1l, W1r, att1, b1,
           W2l, W2r, att2, b2):
    n_s = x_s.shape[0]
    n_t = x_t.shape[0]
    x = jnp.concatenate([x_s, x_t], axis=0)
    srcb = src_batch.astype(jnp.int32)
    refb = ref_batch.astype(jnp.int32)
    kb = _tile_bounds(srcb, refb, n_t, n_s)
    h = _layer(x, W1l, W1r, att1, b1, srcb, refb, kb,
               heads=att1.shape[0], apply_elu=True)
    out = _layer(h, W2l, W2r, att2, b2, srcb, refb, kb,
                 heads=att2.shape[0], apply_elu=False)
    return out[:n_s], out[n_s:]


# speedup vs baseline: 1.2004x; 1.2004x over previous
import functools

import jax
import jax.numpy as jnp
from jax.experimental import pallas as pl
from jax.experimental.pallas import tpu as pltpu

_TILE = 128
_SRC = 32
_NEG = -1e30


def _attn_body(x_ref, wl_ref, wr_ref, att_ref, bias_ref, srcb_ref, refb_ref,
               kb_ref, out_ref, xl_ref, xr_ref, *, heads, out_ch, n_half,
               apply_elu, nbounds):
    i = pl.program_id(0)
    n_tiles = n_half // _TILE

    @pl.when(i == 0)
    def _project():
        xx = x_ref[...]
        xl_ref[...] = jnp.dot(xx, wl_ref[...],
                              preferred_element_type=jnp.float32)
        xr_ref[...] = jnp.dot(xx, wr_ref[...],
                              preferred_element_type=jnp.float32)

    @pl.when(i > 0)
    def _attend():
        idx = i - 1
        dirn = idx // n_tiles
        j = idx % n_tiles
        dst_base = jnp.where(dirn == 0, n_half + j * _TILE, j * _TILE)
        src_off = jnp.where(dirn == 0, 0, n_half)

        t_row = jax.lax.broadcasted_iota(jnp.int32, (1, _TILE), 1) + j * _TILE
        cnt_t = jnp.zeros((1, _TILE), jnp.int32)
        for b in range(nbounds):
            dstb = jnp.where(dirn == 0, refb_ref[b], srcb_ref[b])
            cnt_t = cnt_t + jnp.where(dstb <= t_row, 1, 0)
        bt_row = cnt_t - 1
        valid_t = (bt_row >= 0) & (bt_row < nbounds - 1)

        b_tile = xr_ref[pl.ds(dst_base, _TILE), :]

        k0 = kb_ref[dirn, j, 0]
        k1 = kb_ref[dirn, j, 1]

        def step(k, carry):
            s_base = k * _SRC
            a_full = xl_ref[pl.ds(src_off + s_base, _SRC), :]
            s_col = (jax.lax.broadcasted_iota(jnp.int32, (_SRC, 1), 0)
                     + s_base)
            cnt_s = jnp.zeros((_SRC, 1), jnp.int32)
            for b in range(nbounds):
                srcb = jnp.where(dirn == 0, srcb_ref[b], refb_ref[b])
                cnt_s = cnt_s + jnp.where(srcb <= s_col, 1, 0)
            bs_col = cnt_s - 1
            valid_s = (bs_col >= 0) & (bs_col < nbounds - 1)
            mask = (bs_col == bt_row) & valid_s & valid_t
            maskf = mask.astype(jnp.float32)

            new_carry = []
            for h in range(heads):
                m, d, acc = carry[h]
                a = a_full[:, h * out_ch:(h + 1) * out_ch]
                bb = b_tile[:, h * out_ch:(h + 1) * out_ch]
                att = att_ref[h:h + 1, :].reshape(1, 1, out_ch)
                e = a[:, None, :] + bb[None, :, :]
                e = jnp.maximum(e, 0.2 * e) * att
                logit = jnp.sum(e, axis=-1)
                lm = jnp.where(mask, logit, _NEG)
                tmax = jnp.max(lm, axis=0, keepdims=True)
                m_new = jnp.maximum(m, tmax)
                corr = jnp.exp(m - m_new)
                p = jnp.exp(lm - m_new) * maskf
                d_new = d * corr + jnp.sum(p, axis=0, keepdims=True)
                pv = jax.lax.dot_general(
                    p, a, (((0,), (0,)), ((), ())),
                    preferred_element_type=jnp.float32)
                acc_new = acc * corr.reshape(_TILE, 1) + pv
                new_carry.append((m_new, d_new, acc_new))
            return tuple(new_carry)

        init = tuple((jnp.full((1, _TILE), _NEG, jnp.float32),
                      jnp.zeros((1, _TILE), jnp.float32),
                      jnp.zeros((_TILE, out_ch), jnp.float32))
                     for _ in range(heads))
        carry = jax.lax.fori_loop(k0, k1 + 1, step, init)

        outs = []
        for h in range(heads):
            m, d, acc = carry[h]
            den = jnp.maximum(d.reshape(_TILE, 1), 1e-16)
            outs.append(acc / den)
        val = jnp.concatenate(outs, axis=1) + bias_ref[...]
        if apply_elu:
            val = jnp.where(val > 0, val, jnp.exp(jnp.minimum(val, 0.0)) - 1.0)
        out_ref[pl.ds(dst_base, _TILE), :] = val


def _layer(x, wl, wr, att, bias, srcb, refb, kb, heads, apply_elu):
    n2, _ = x.shape
    hc = wl.shape[1]
    out_ch = hc // heads
    n_half = n2 // 2
    n_tiles = n_half // _TILE
    grid = (1 + 2 * n_tiles,)
    body = functools.partial(
        _attn_body, heads=heads, out_ch=out_ch, n_half=n_half,
        apply_elu=apply_elu, nbounds=srcb.shape[0])
    vmem = pl.BlockSpec(memory_space=pltpu.VMEM)
    smem = pl.BlockSpec(memory_space=pltpu.SMEM)
    return pl.pallas_call(
        body,
        grid=grid,
        in_specs=[vmem, vmem, vmem, vmem, vmem, smem, smem, smem],
        out_specs=vmem,
        out_shape=jax.ShapeDtypeStruct((n2, hc), jnp.float32),
        scratch_shapes=[pltpu.VMEM((n2, hc), jnp.float32),
                        pltpu.VMEM((n2, hc), jnp.float32)],
    )(x, wl, wr, att, bias.reshape(1, hc), srcb, refb, kb)


def _tile_bounds(srcb, refb, n_dst, n_src):
    nb = srcb.shape[0] - 1

    def one(dstb, sb):
        idxn = jnp.arange(n_dst, dtype=jnp.int32)
        bt = jnp.searchsorted(dstb, idxn, side='right').astype(jnp.int32) - 1
        valid = (bt >= 0) & (bt < nb)
        btt = bt.reshape(-1, _TILE)
        vt = valid.reshape(-1, _TILE)
        blo = jnp.min(jnp.where(vt, btt, nb), axis=1)
        bhi = jnp.max(jnp.where(vt, btt, -1), axis=1)
        any_valid = jnp.any(vt, axis=1)
        s_lo = sb[jnp.clip(blo, 0, nb)]
        s_hi = sb[jnp.clip(bhi + 1, 0, nb)]
        k0 = s_lo // _SRC
        k1 = (s_hi + _SRC - 1) // _SRC - 1
        empty = (~any_valid) | (s_hi <= s_lo)
        k0 = jnp.where(empty, 1, k0)
        k1 = jnp.where(empty, 0, k1)
        return jnp.stack([k0, k1], axis=1)

    return jnp.stack([one(refb, srcb), one(srcb, refb)], 0).astype(jnp.int32)


def kernel(x_s, x_t, src_batch, ref_batch, W1l, W1r, att1, b1,
           W2l, W2r, att2, b2):
    n_s = x_s.shape[0]
    n_t = x_t.shape[0]
    x = jnp.concatenate([x_s, x_t], axis=0)
    srcb = src_batch.astype(jnp.int32)
    refb = ref_batch.astype(jnp.int32)
    kb = _tile_bounds(srcb, refb, n_t, n_s)
    h = _layer(x, W1l, W1r, att1, b1, srcb, refb, kb,
               heads=att1.shape[0], apply_elu=True)
    out = _layer(h, W2l, W2r, att2, b2, srcb, refb, kb,
                 heads=att2.shape[0], apply_elu=False)
    return out[:n_s], out[n_s:]


# speedup vs baseline: 1.2525x; 1.0434x over previous
import functools

import jax
import jax.numpy as jnp
from jax.experimental import pallas as pl
from jax.experimental.pallas import tpu as pltpu

_TILE = 128
_SRC = 32
_NEG = -1e30


def _attn_body(x_ref, wl_ref, wr_ref, attw_ref, sgn_ref, bias_ref, srcb_ref,
               refb_ref, kb_ref, out_ref, xl_ref, xr_ref, *, heads, out_ch,
               n_half, apply_elu, nbounds):
    i = pl.program_id(0)
    n_tiles = n_half // _TILE

    @pl.when(i == 0)
    def _project():
        xx = x_ref[...]
        xl_ref[...] = jnp.dot(xx, wl_ref[...],
                              preferred_element_type=jnp.float32)
        xr_ref[...] = jnp.dot(xx, wr_ref[...],
                              preferred_element_type=jnp.float32)

    @pl.when(i > 0)
    def _attend():
        idx = i - 1
        dirn = idx // n_tiles
        j = idx % n_tiles
        dst_base = jnp.where(dirn == 0, n_half + j * _TILE, j * _TILE)
        src_off = jnp.where(dirn == 0, 0, n_half)

        t_row = jax.lax.broadcasted_iota(jnp.int32, (1, _TILE), 1) + j * _TILE
        cnt_t = jnp.zeros((1, _TILE), jnp.int32)
        for b in range(nbounds):
            dstb = jnp.where(dirn == 0, refb_ref[b], srcb_ref[b])
            cnt_t = cnt_t + jnp.where(dstb <= t_row, 1, 0)
        bt_row = cnt_t - 1
        valid_t = (bt_row >= 0) & (bt_row < nbounds - 1)

        b_tile = xr_ref[pl.ds(dst_base, _TILE), :]

        bw_all = (b_tile * attw_ref[...]).T[:, None, :]
        sg_all = sgn_ref[...][:, :, None]
        bw3 = [bw_all[h * out_ch:(h + 1) * out_ch] for h in range(heads)]
        sg3 = [sg_all[h * out_ch:(h + 1) * out_ch] for h in range(heads)]

        k0 = kb_ref[dirn, j, 0]
        k1 = kb_ref[dirn, j, 1]

        def step(k, carry):
            s_base = k * _SRC
            a_full = xl_ref[pl.ds(src_off + s_base, _SRC), :]
            s_col = (jax.lax.broadcasted_iota(jnp.int32, (_SRC, 1), 0)
                     + s_base)
            cnt_s = jnp.zeros((_SRC, 1), jnp.int32)
            for b in range(nbounds):
                srcb = jnp.where(dirn == 0, srcb_ref[b], refb_ref[b])
                cnt_s = cnt_s + jnp.where(srcb <= s_col, 1, 0)
            bs_col = cnt_s - 1
            valid_s = (bs_col >= 0) & (bs_col < nbounds - 1)
            mask = (bs_col == bt_row) & valid_s & valid_t
            maskf = mask.astype(jnp.float32)

            new_carry = []
            aw_all = (a_full * attw_ref[...]).T[:, :, None]
            for h in range(heads):
                m, d, acc = carry[h]
                a = a_full[:, h * out_ch:(h + 1) * out_ch]
                aw3 = aw_all[h * out_ch:(h + 1) * out_ch]
                u = aw3 + bw3[h]
                u = u + sg3[h] * jnp.abs(u)
                logit = jnp.sum(u, axis=0)
                lm = jnp.where(mask, logit, _NEG)
                tmax = jnp.max(lm, axis=0, keepdims=True)
                m_new = jnp.maximum(m, tmax)
                corr = jnp.exp(m - m_new)
                p = jnp.exp(lm - m_new) * maskf
                d_new = d * corr + jnp.sum(p, axis=0, keepdims=True)
                pv = jax.lax.dot_general(
                    p, a, (((0,), (0,)), ((), ())),
                    preferred_element_type=jnp.float32)
                acc_new = acc * corr.reshape(_TILE, 1) + pv
                new_carry.append((m_new, d_new, acc_new))
            return tuple(new_carry)

        init = tuple((jnp.full((1, _TILE), _NEG, jnp.float32),
                      jnp.zeros((1, _TILE), jnp.float32),
                      jnp.zeros((_TILE, out_ch), jnp.float32))
                     for _ in range(heads))
        carry = jax.lax.fori_loop(k0, k1 + 1, step, init)

        outs = []
        for h in range(heads):
            m, d, acc = carry[h]
            den = jnp.maximum(d.reshape(_TILE, 1), 1e-16)
            outs.append(acc / den)
        val = jnp.concatenate(outs, axis=1) + bias_ref[...]
        if apply_elu:
            val = jnp.where(val > 0, val, jnp.exp(jnp.minimum(val, 0.0)) - 1.0)
        out_ref[pl.ds(dst_base, _TILE), :] = val


def _layer(x, wl, wr, att, bias, srcb, refb, kb, heads, apply_elu):
    n2, _ = x.shape
    hc = wl.shape[1]
    out_ch = hc // heads
    n_half = n2 // 2
    n_tiles = n_half // _TILE
    grid = (1 + 2 * n_tiles,)
    body = functools.partial(
        _attn_body, heads=heads, out_ch=out_ch, n_half=n_half,
        apply_elu=apply_elu, nbounds=srcb.shape[0])
    attw = (0.6 * att).reshape(1, hc)
    sgnT = ((2.0 / 3.0) * jnp.sign(att)).reshape(hc, 1)
    vmem = pl.BlockSpec(memory_space=pltpu.VMEM)
    smem = pl.BlockSpec(memory_space=pltpu.SMEM)
    return pl.pallas_call(
        body,
        grid=grid,
        in_specs=[vmem, vmem, vmem, vmem, vmem, vmem, smem, smem, smem],
        out_specs=vmem,
        out_shape=jax.ShapeDtypeStruct((n2, hc), jnp.float32),
        scratch_shapes=[pltpu.VMEM((n2, hc), jnp.float32),
                        pltpu.VMEM((n2, hc), jnp.float32)],
    )(x, wl, wr, attw, sgnT, bias.reshape(1, hc), srcb, refb, kb)


def _tile_bounds(srcb, refb, n_dst, n_src):
    nb = srcb.shape[0] - 1

    def one(dstb, sb):
        idxn = jnp.arange(n_dst, dtype=jnp.int32)
        bt = jnp.searchsorted(dstb, idxn, side='right').astype(jnp.int32) - 1
        valid = (bt >= 0) & (bt < nb)
        btt = bt.reshape(-1, _TILE)
        vt = valid.reshape(-1, _TILE)
        blo = jnp.min(jnp.where(vt, btt, nb), axis=1)
        bhi = jnp.max(jnp.where(vt, btt, -1), axis=1)
        any_valid = jnp.any(vt, axis=1)
        s_lo = sb[jnp.clip(blo, 0, nb)]
        s_hi = sb[jnp.clip(bhi + 1, 0, nb)]
        k0 = s_lo // _SRC
        k1 = (s_hi + _SRC - 1) // _SRC - 1
        empty = (~any_valid) | (s_hi <= s_lo)
        k0 = jnp.where(empty, 1, k0)
        k1 = jnp.where(empty, 0, k1)
        return jnp.stack([k0, k1], axis=1)

    return jnp.stack([one(refb, srcb), one(srcb, refb)], 0).astype(jnp.int32)


def kernel(x_s, x_t, src_batch, ref_batch, W1l, W1r, att1, b1,
           W2l, W2r, att2, b2):
    n_s = x_s.shape[0]
    n_t = x_t.shape[0]
    x = jnp.concatenate([x_s, x_t], axis=0)
    srcb = src_batch.astype(jnp.int32)
    refb = ref_batch.astype(jnp.int32)
    kb = _tile_bounds(srcb, refb, n_t, n_s)
    h = _layer(x, W1l, W1r, att1, b1, srcb, refb, kb,
               heads=att1.shape[0], apply_elu=True)
    out = _layer(h, W2l, W2r, att2, b2, srcb, refb, kb,
                 heads=att2.shape[0], apply_elu=False)
    return out[:n_s], out[n_s:]
